# baseline (speedup 1.0000x reference)
import functools

import jax
import jax.numpy as jnp
from jax import lax
from jax.experimental import pallas as pl
from jax.experimental.pallas import tpu as pltpu
from jax.experimental.pallas import tpu_sc as plsc

N = 10000
E = 320000
D = 128
DH = D // 2
NC = 2
NS = 16
K = 128
NBUF = 2
CH = 160
SCH = CH
E_PAD = NS * CH * K
N_PAD = 10240
RPT = N_PAD // NS


def _segsum_body(x_hbm, src_hbm, dst_hbm, out_hbm,
                 src_v, dst_v, rows_v, zbuf, agg_sh, *gsem):
    cid = lax.axis_index("c")
    sid = lax.axis_index("s")

    pltpu.sync_copy(src_hbm.at[sid], src_v)
    pltpu.sync_copy(dst_hbm.at[sid], dst_v)

    zeros16 = jnp.zeros((16,), jnp.float32)

    def _zrow(i, carry):
        for j in range(DH // 16):
            zbuf[i, pl.ds(j * 16, 16)] = zeros16
        return carry

    lax.fori_loop(0, K, _zrow, None)

    table = x_hbm.at[cid]

    def _gather(j, b, sem):
        pltpu.async_copy(table.at[src_v.at[j]], rows_v.at[b], sem)

    def _wait_gather(j, b, sem):
        pltpu.make_async_copy(table.at[src_v.at[j]], rows_v.at[b], sem).wait()

    def _scatter(j, b):
        pltpu.sync_copy(rows_v.at[b], agg_sh.at[dst_v.at[j]], add=True)

    for b in range(NBUF):
        _gather(b, b, gsem[b])

    for kk in range(RPT // K):
        pltpu.sync_copy(zbuf, agg_sh.at[pl.ds(sid * RPT + kk * K, K)])

    plsc.subcore_barrier()

    def _chunk(i, carry):
        j0 = i * NBUF
        for u in range(NBUF):
            j = j0 + u
            _wait_gather(j, u, gsem[u])
            _scatter(j, u)

            @pl.when(j + NBUF < SCH)
            def _():
                _gather(j + NBUF, u, gsem[u])

        return carry

    lax.fori_loop(0, SCH // NBUF, _chunk, None)

    plsc.subcore_barrier()

    base = sid * RPT

    @pl.when(sid < NS - 1)
    def _():
        pltpu.sync_copy(agg_sh.at[pl.ds(base, RPT)],
                        out_hbm.at[cid, pl.ds(base, RPT)])

    @pl.when(sid == NS - 1)
    def _():
        last = N - (NS - 1) * RPT
        pltpu.sync_copy(agg_sh.at[pl.ds(base, last)],
                        out_hbm.at[cid, pl.ds(base, last)])


_segsum = pl.kernel(
    _segsum_body,
    mesh=plsc.VectorSubcoreMesh(core_axis_name="c", subcore_axis_name="s"),
    out_type=jax.ShapeDtypeStruct((NC, N, DH), jnp.float32),
    scratch_types=[
        pltpu.VMEM((CH, K), jnp.int32),
        pltpu.VMEM((CH, K), jnp.int32),
        pltpu.VMEM((NBUF, K, DH), jnp.float32),
        pltpu.VMEM((K, DH), jnp.float32),
        pltpu.VMEM_SHARED((N_PAD, DH), jnp.float32),
    ] + [pltpu.SemaphoreType.DMA] * NBUF,
    compiler_params=pltpu.CompilerParams(use_tc_tiling_on_sc=False),
)


def _dense_body(p_ref, x_ref, wrel_ref, wroot_ref, b_ref, o_ref, *,
                apply_relu, split_out):
    agg = jnp.concatenate([p_ref[0], p_ref[1]], axis=1)
    acc = lax.dot(agg, wrel_ref[...], precision=lax.Precision.HIGHEST,
                  preferred_element_type=jnp.float32)
    xin = jnp.concatenate([x_ref[0], x_ref[1]], axis=1)
    acc = acc + lax.dot(xin, wroot_ref[...], precision=lax.Precision.HIGHEST,
                        preferred_element_type=jnp.float32)
    acc = acc + b_ref[...]
    if apply_relu:
        acc = jnp.maximum(acc, 0.0)
    if split_out:
        o_ref[0] = acc[:, :DH]
        o_ref[1] = acc[:, DH:]
    else:
        o_ref[...] = acc


def _make_dense(apply_relu, split_out):
    blk = 1000
    if split_out:
        out_shape = jax.ShapeDtypeStruct((NC, N, DH), jnp.float32)
        out_specs = pl.BlockSpec((NC, blk, DH), lambda i: (0, i, 0))
    else:
        out_shape = jax.ShapeDtypeStruct((N, D), jnp.float32)
        out_specs = pl.BlockSpec((blk, D), lambda i: (i, 0))
    return pl.pallas_call(
        functools.partial(_dense_body, apply_relu=apply_relu,
                          split_out=split_out),
        grid=(N // blk,),
        in_specs=[
            pl.BlockSpec((NC, blk, DH), lambda i: (0, i, 0)),
            pl.BlockSpec((NC, blk, DH), lambda i: (0, i, 0)),
            pl.BlockSpec((D, D), lambda i: (0, 0)),
            pl.BlockSpec((D, D), lambda i: (0, 0)),
            pl.BlockSpec((1, D), lambda i: (0, 0)),
        ],
        out_specs=out_specs,
        out_shape=out_shape,
    )


_dense_relu_split = _make_dense(True, True)
_dense_plain = _make_dense(False, False)


@jax.jit
def kernel(x, edge_index, W1_rel, b1, W1_root, W2_rel, b2, W2_root):
    src = edge_index[0].astype(jnp.int32)
    dst = edge_index[1].astype(jnp.int32)
    pad = E_PAD - E
    src_p = jnp.concatenate([src, jnp.zeros((pad,), jnp.int32)]).reshape(NS, CH, K)
    pad_dst = N + (jnp.arange(pad, dtype=jnp.int32) % (N_PAD - N))
    dst_p = jnp.concatenate([dst, pad_dst]).reshape(NS, CH, K)

    x_split = jnp.stack([x[:, :DH], x[:, DH:]])
    p1 = _segsum(x_split, src_p, dst_p)
    h_split = _dense_relu_split(p1, x_split, W1_rel, W1_root, b1.reshape(1, D))
    p2 = _segsum(h_split, src_p, dst_p)
    out = _dense_plain(p2, h_split, W2_rel, W2_root, b2.reshape(1, D))
    return out


# speedup vs baseline: 1.4286x; 1.4286x over previous
import functools

import jax
import jax.numpy as jnp
from jax import lax
from jax.experimental import pallas as pl
from jax.experimental.pallas import tpu as pltpu
from jax.experimental.pallas import tpu_sc as plsc

N = 10000
E = 320000
D = 128
DH = D // 2
NC = 2
NS = 16
K = 128
NBUF = 2
CH = 158
SCH = CH
E_PAD = NS * CH * K
N_PAD = 10240
RPT = N_PAD // NS


def _segsum_body(x_hbm, src_hbm, dst_hbm, out_hbm,
                 src_v, dst_v, rows_v, zbuf, agg_sh, *gsem):
    cid = lax.axis_index("c")
    sid = lax.axis_index("s")

    pltpu.sync_copy(src_hbm.at[sid], src_v)
    pltpu.sync_copy(dst_hbm.at[sid], dst_v)

    zeros16 = jnp.zeros((16,), jnp.float32)

    def _zrow(i, carry):
        for j in range(DH // 16):
            zbuf[i, pl.ds(j * 16, 16)] = zeros16
        return carry

    lax.fori_loop(0, K, _zrow, None)

    table = x_hbm.at[cid]

    def _gather(j, b, sem):
        pltpu.async_copy(table.at[src_v.at[j]], rows_v.at[b], sem)

    def _wait_gather(j, b, sem):
        pltpu.make_async_copy(table.at[src_v.at[j]], rows_v.at[b], sem).wait()

    def _scatter(j, b):
        pltpu.sync_copy(rows_v.at[b], agg_sh.at[dst_v.at[j]], add=True)

    for b in range(NBUF):
        _gather(b, b, gsem[b])

    for kk in range(RPT // K):
        pltpu.sync_copy(zbuf, agg_sh.at[pl.ds(sid * RPT + kk * K, K)])

    plsc.subcore_barrier()

    def _chunk(i, carry):
        j0 = i * NBUF
        for u in range(NBUF):
            j = j0 + u
            _wait_gather(j, u, gsem[u])
            _scatter(j, u)

            @pl.when(j + NBUF < SCH)
            def _():
                _gather(j + NBUF, u, gsem[u])

        return carry

    lax.fori_loop(0, SCH // NBUF, _chunk, None)

    plsc.subcore_barrier()

    base = sid * RPT

    @pl.when(sid < NS - 1)
    def _():
        pltpu.sync_copy(agg_sh.at[pl.ds(base, RPT)],
                        out_hbm.at[cid, pl.ds(base, RPT)])

    @pl.when(sid == NS - 1)
    def _():
        last = N - (NS - 1) * RPT
        pltpu.sync_copy(agg_sh.at[pl.ds(base, last)],
                        out_hbm.at[cid, pl.ds(base, last)])


_segsum = pl.kernel(
    _segsum_body,
    mesh=plsc.VectorSubcoreMesh(core_axis_name="c", subcore_axis_name="s"),
    out_type=jax.ShapeDtypeStruct((NC, N, DH), jnp.float32),
    scratch_types=[
        pltpu.VMEM((CH, K), jnp.int32),
        pltpu.VMEM((CH, K), jnp.int32),
        pltpu.VMEM((NBUF, K, DH), jnp.float32),
        pltpu.VMEM((K, DH), jnp.float32),
        pltpu.VMEM_SHARED((N_PAD, DH), jnp.float32),
    ] + [pltpu.SemaphoreType.DMA] * NBUF,
    compiler_params=pltpu.CompilerParams(use_tc_tiling_on_sc=False),
)


def _dense_body(p_ref, x_ref, wrel_ref, wroot_ref, b_ref, o_ref, *,
                apply_relu, split_out):
    agg = jnp.concatenate([p_ref[0], p_ref[1]], axis=1)
    acc = lax.dot(agg, wrel_ref[...], precision=lax.Precision.HIGHEST,
                  preferred_element_type=jnp.float32)
    xin = jnp.concatenate([x_ref[0], x_ref[1]], axis=1)
    acc = acc + lax.dot(xin, wroot_ref[...], precision=lax.Precision.HIGHEST,
                        preferred_element_type=jnp.float32)
    acc = acc + b_ref[...]
    if apply_relu:
        acc = jnp.maximum(acc, 0.0)
    if split_out:
        o_ref[0] = acc[:, :DH]
        o_ref[1] = acc[:, DH:]
    else:
        o_ref[...] = acc


def _make_dense(apply_relu, split_out):
    blk = 1000
    if split_out:
        out_shape = jax.ShapeDtypeStruct((NC, N, DH), jnp.float32)
        out_specs = pl.BlockSpec((NC, blk, DH), lambda i: (0, i, 0))
    else:
        out_shape = jax.ShapeDtypeStruct((N, D), jnp.float32)
        out_specs = pl.BlockSpec((blk, D), lambda i: (i, 0))
    return pl.pallas_call(
        functools.partial(_dense_body, apply_relu=apply_relu,
                          split_out=split_out),
        grid=(N // blk,),
        in_specs=[
            pl.BlockSpec((NC, blk, DH), lambda i: (0, i, 0)),
            pl.BlockSpec((NC, blk, DH), lambda i: (0, i, 0)),
            pl.BlockSpec((D, D), lambda i: (0, 0)),
            pl.BlockSpec((D, D), lambda i: (0, 0)),
            pl.BlockSpec((1, D), lambda i: (0, 0)),
        ],
        out_specs=out_specs,
        out_shape=out_shape,
    )


_dense_relu_split = _make_dense(True, True)
_dense_plain = _make_dense(False, False)


@jax.jit
def kernel(x, edge_index, W1_rel, b1, W1_root, W2_rel, b2, W2_root):
    src = edge_index[0].astype(jnp.int32)
    dst = edge_index[1].astype(jnp.int32)
    pad = E_PAD - E
    src_p = jnp.concatenate([src, jnp.zeros((pad,), jnp.int32)]).reshape(NS, CH, K)
    pad_dst = N + (jnp.arange(pad, dtype=jnp.int32) % (N_PAD - N))
    dst_p = jnp.concatenate([dst, pad_dst]).reshape(NS, CH, K)

    x_split = jnp.stack([x[:, :DH], x[:, DH:]])
    p1 = _segsum(x_split, src_p, dst_p)
    h_split = _dense_relu_split(p1, x_split, W1_rel, W1_root, b1.reshape(1, D))
    p2 = _segsum(h_split, src_p, dst_p)
    out = _dense_plain(p2, h_split, W2_rel, W2_root, b2.reshape(1, D))
    return out


# speedup vs baseline: 2.5341x; 1.7739x over previous
import functools

import jax
import jax.numpy as jnp
from jax import lax
from jax.experimental import pallas as pl
from jax.experimental.pallas import tpu as pltpu
from jax.experimental.pallas import tpu_sc as plsc

N = 10000
E = 320000
D = 128
DH = D // 2
NC = 2
NS = 16
K = 128
NBUF = 4
CH = 160
SCH = CH
E_PAD = NS * CH * K
N_PAD = 10240
RPT = N_PAD // NS


def _segsum_body(x_hbm, src_hbm, dst_hbm, out_hbm,
                 src_v, dst_v, rows_v, zbuf, agg_sh, *gsem):
    cid = lax.axis_index("c")
    sid = lax.axis_index("s")

    pltpu.sync_copy(src_hbm.at[sid], src_v)
    pltpu.sync_copy(dst_hbm.at[sid], dst_v)

    zeros16 = jnp.zeros((16,), jnp.float32)

    def _zrow(i, carry):
        for j in range(DH // 16):
            zbuf[i, pl.ds(j * 16, 16)] = zeros16
        return carry

    lax.fori_loop(0, K, _zrow, None)

    table = x_hbm.at[cid]

    def _gather(j, b, sem):
        pltpu.async_copy(table.at[src_v.at[j]], rows_v.at[b], sem)

    def _wait_gather(j, b, sem):
        pltpu.make_async_copy(table.at[src_v.at[j]], rows_v.at[b], sem).wait()

    def _scatter(j, b):
        pltpu.sync_copy(rows_v.at[b], agg_sh.at[dst_v.at[j]], add=True)

    for b in range(NBUF):
        _gather(b, b, gsem[b])

    for kk in range(RPT // K):
        pltpu.sync_copy(zbuf, agg_sh.at[pl.ds(sid * RPT + kk * K, K)])

    plsc.subcore_barrier()

    def _chunk(i, carry):
        j0 = i * NBUF
        for u in range(NBUF):
            j = j0 + u
            _wait_gather(j, u, gsem[u])
            _scatter(j, u)

            @pl.when(j + NBUF < SCH)
            def _():
                _gather(j + NBUF, u, gsem[u])

        return carry

    lax.fori_loop(0, SCH // NBUF, _chunk, None)

    plsc.subcore_barrier()

    base = sid * RPT

    @pl.when(sid < NS - 1)
    def _():
        pltpu.sync_copy(agg_sh.at[pl.ds(base, RPT)],
                        out_hbm.at[cid, pl.ds(base, RPT)])

    @pl.when(sid == NS - 1)
    def _():
        last = N - (NS - 1) * RPT
        pltpu.sync_copy(agg_sh.at[pl.ds(base, last)],
                        out_hbm.at[cid, pl.ds(base, last)])


_segsum = pl.kernel(
    _segsum_body,
    mesh=plsc.VectorSubcoreMesh(core_axis_name="c", subcore_axis_name="s"),
    out_type=jax.ShapeDtypeStruct((NC, N, DH), jnp.float32),
    scratch_types=[
        pltpu.VMEM((CH, K), jnp.int32),
        pltpu.VMEM((CH, K), jnp.int32),
        pltpu.VMEM((NBUF, K, DH), jnp.float32),
        pltpu.VMEM((K, DH), jnp.float32),
        pltpu.VMEM_SHARED((N_PAD, DH), jnp.float32),
    ] + [pltpu.SemaphoreType.DMA] * NBUF,
    compiler_params=pltpu.CompilerParams(use_tc_tiling_on_sc=False),
)


def _dense_body(p_ref, x_ref, wrel_ref, wroot_ref, b_ref, o_ref, *,
                apply_relu, split_out):
    agg = jnp.concatenate([p_ref[0], p_ref[1]], axis=1)
    acc = lax.dot(agg, wrel_ref[...], precision=lax.Precision.HIGHEST,
                  preferred_element_type=jnp.float32)
    xin = jnp.concatenate([x_ref[0], x_ref[1]], axis=1)
    acc = acc + lax.dot(xin, wroot_ref[...], precision=lax.Precision.HIGHEST,
                        preferred_element_type=jnp.float32)
    acc = acc + b_ref[...]
    if apply_relu:
        acc = jnp.maximum(acc, 0.0)
    if split_out:
        o_ref[0] = acc[:, :DH]
        o_ref[1] = acc[:, DH:]
    else:
        o_ref[...] = acc


def _make_dense(apply_relu, split_out):
    blk = 1000
    if split_out:
        out_shape = jax.ShapeDtypeStruct((NC, N, DH), jnp.float32)
        out_specs = pl.BlockSpec((NC, blk, DH), lambda i: (0, i, 0))
    else:
        out_shape = jax.ShapeDtypeStruct((N, D), jnp.float32)
        out_specs = pl.BlockSpec((blk, D), lambda i: (i, 0))
    return pl.pallas_call(
        functools.partial(_dense_body, apply_relu=apply_relu,
                          split_out=split_out),
        grid=(N // blk,),
        in_specs=[
            pl.BlockSpec((NC, blk, DH), lambda i: (0, i, 0)),
            pl.BlockSpec((NC, blk, DH), lambda i: (0, i, 0)),
            pl.BlockSpec((D, D), lambda i: (0, 0)),
            pl.BlockSpec((D, D), lambda i: (0, 0)),
            pl.BlockSpec((1, D), lambda i: (0, 0)),
        ],
        out_specs=out_specs,
        out_shape=out_shape,
    )


_dense_relu_split = _make_dense(True, True)
_dense_plain = _make_dense(False, False)


@jax.jit
def kernel(x, edge_index, W1_rel, b1, W1_root, W2_rel, b2, W2_root):
    src = edge_index[0].astype(jnp.int32)
    dst = edge_index[1].astype(jnp.int32)
    pad = E_PAD - E
    pad_src = jnp.arange(pad, dtype=jnp.int32) % N
    src_p = jnp.concatenate([src, pad_src]).reshape(NS, CH, K)
    pad_dst = N + (jnp.arange(pad, dtype=jnp.int32) % (N_PAD - N))
    dst_p = jnp.concatenate([dst, pad_dst]).reshape(NS, CH, K)

    x_split = jnp.stack([x[:, :DH], x[:, DH:]])
    p1 = _segsum(x_split, src_p, dst_p)
    h_split = _dense_relu_split(p1, x_split, W1_rel, W1_root, b1.reshape(1, D))
    p2 = _segsum(h_split, src_p, dst_p)
    out = _dense_plain(p2, h_split, W2_rel, W2_root, b2.reshape(1, D))
    return out
